# Initial kernel scaffold; baseline (speedup 1.0000x reference)
import jax
import jax.numpy as jnp
from jax.experimental import pallas as pl


def kernel(input_ids, state, embed, sr_Wk, sr_Wv, sr_Wq, sr_Wbw, sr_Wbb, sr_Waw, sr_Wab, sr_Wout, sr_nw, sr_gw, sw_Wk, sw_Wv, sw_Wq, sw_Wbw, sw_Wbb, sw_Waw, sw_Wab, sw_Wout, sw_nw, sw_gw, ln1_w, wq, wk, wv, wo, ln2_w, w_gate, w_up, w_down, final_norm_w):
    raise NotImplementedError("write your pallas kernel here")


# speedup vs baseline: 7.8527x; 7.8527x over previous
import numpy as np
import jax
import jax.numpy as jnp
from jax.experimental import pallas as pl
from jax.experimental.pallas import tpu as pltpu

HID = 512; NH = 8; HD = 64
SH = 8; SD = 32; SDIM = 256
NL = 6; FFN = 1536; VOC = 63
THETA = 500000.0; EPS = 1e-6
B = 4; L = 1024
TL = 512
VOCP = 128
LQ = L // 8
f32 = jnp.float32

_VMEM = 50 * 1024 * 1024


def _cp(*sem):
    return pltpu.CompilerParams(dimension_semantics=sem,
                                vmem_limit_bytes=_VMEM)


def _rope_np():
    inv_freq = 1.0 / THETA ** (np.arange(0, HD, 2, dtype=np.float32) / HD)
    t = np.arange(L, dtype=np.float32)
    freqs = np.outer(t, inv_freq)
    emb = np.concatenate([freqs, freqs], axis=-1)
    cos = np.tile(np.cos(emb), (1, NH)).astype(np.float32)
    sin = np.tile(np.sin(emb), (1, NH)).astype(np.float32)
    return cos, sin


_COS_NP, _SIN_NP = _rope_np()
_MASKB_NP = np.where(np.tril(np.ones((L, L), bool)), 0.0,
                     -1e9).astype(np.float32)
_MONES_NP = np.kron(np.eye(SH), np.ones((SD, SD))).astype(np.float32)
_E16_NP = np.kron(np.eye(16), np.ones((1, SD))).astype(np.float32)


def _rms(x, w):
    return x * jax.lax.rsqrt(jnp.mean(x * x, axis=-1, keepdims=True) + EPS) * w


def _mmT(x, w):
    return jax.lax.dot_general(x, w, (((1,), (1,)), ((), ())),
                               preferred_element_type=f32)


def _embed_body(ids_ref, emb_ref, out_ref):
    ids = ids_ref[...].reshape(1, TL)
    iota = jax.lax.broadcasted_iota(jnp.int32, (VOCP, TL), 0)
    oh = jnp.where(jnp.broadcast_to(ids, (VOCP, TL)) == iota,
                   f32(1.0), f32(0.0))
    h0 = jax.lax.dot_general(oh, emb_ref[...], (((0,), (0,)), ((), ())),
                             preferred_element_type=f32)
    out_ref[...] = h0.reshape(1, TL, HID)


def _embed_call(ids_f, embed_pad):
    return pl.pallas_call(
        _embed_body,
        grid=(B, L // TL),
        in_specs=[pl.BlockSpec((1, 1, TL), lambda b, t: (b, 0, t)),
                  pl.BlockSpec((VOCP, HID), lambda b, t: (0, 0))],
        out_specs=pl.BlockSpec((1, TL, HID), lambda b, t: (b, t, 0)),
        out_shape=jax.ShapeDtypeStruct((B, L, HID), f32),
        compiler_params=_cp("parallel", "parallel"),
    )(ids_f, embed_pad)


def _proj_body(x_ref, w_ref, bb_ref, ab_ref, mo_ref,
               k_ref, v_ref, q_ref, b_ref, a_ref, g_ref):
    x = x_ref[0]
    p = _mmT(x, w_ref[...])
    k = p[:, 0:256]; v = p[:, 256:512]; q = p[:, 512:768]
    be = jax.nn.sigmoid(p[:, 768:1024] + bb_ref[...])
    al = jax.nn.sigmoid(p[:, 1024:1280] + ab_ref[...])
    g = p[:, 1280:1536]
    n2 = _mmT(k * k, mo_ref[...])
    kh = k / jnp.maximum(jnp.sqrt(n2), 1e-12)
    for r, val in ((k_ref, kh), (v_ref, v), (q_ref, q),
                   (b_ref, be), (a_ref, al), (g_ref, g)):
        r[...] = val.reshape(1, TL, SDIM)


def _proj_call(h3d, Wcat, bb, ab, mones):
    o = jax.ShapeDtypeStruct((B, L, SDIM), f32)
    ospec = pl.BlockSpec((1, TL, SDIM), lambda b, t: (b, t, 0))
    return pl.pallas_call(
        _proj_body,
        grid=(B, L // TL),
        in_specs=[pl.BlockSpec((1, TL, HID), lambda b, t: (b, t, 0)),
                  pl.BlockSpec((6 * SDIM, HID), lambda b, t: (0, 0)),
                  pl.BlockSpec((1, SDIM), lambda b, t: (0, 0)),
                  pl.BlockSpec((1, SDIM), lambda b, t: (0, 0)),
                  pl.BlockSpec((SDIM, SDIM), lambda b, t: (0, 0))],
        out_specs=[ospec] * 6,
        out_shape=[o] * 6,
        compiler_params=_cp("parallel", "parallel"),
    )(h3d, Wcat, bb, ab, mones)


def _scan_body(a_ref, v_ref, b_ref, k_ref, q_ref, s0_ref, e_ref,
               ctx_ref, sout_ref):
    E = e_ref[...]

    def body(iq, S):
        a8 = a_ref[0, iq]
        v8 = v_ref[0, iq]
        b8 = b_ref[0, iq]
        k8 = k_ref[0, iq]
        q8 = q_ref[0, iq]
        rows = []
        for r in range(8):
            S = S * a8[r:r + 1, :]
            kr = jax.lax.dot_general(k8[:, r * 16:(r + 1) * 16], E,
                                     (((1,), (0,)), ((), ())),
                                     preferred_element_type=f32)
            pred = jnp.sum(S * kr, axis=0, keepdims=True)
            w = b8[r:r + 1, :] * (v8[r:r + 1, :] - pred)
            S = S + kr * w
            qr = jax.lax.dot_general(q8[:, r * 16:(r + 1) * 16], E,
                                     (((1,), (0,)), ((), ())),
                                     preferred_element_type=f32)
            rows.append(jnp.sum(S * qr, axis=0, keepdims=True))
        ctx_ref[0, iq] = jnp.concatenate(rows, axis=0)
        return S

    S = jax.lax.fori_loop(0, LQ, body, s0_ref[...].reshape(SD, 512))
    sout_ref[...] = S.reshape(1, SD, 512)


def _scan_call(a8, v8, b8, k8, q8, s0, e16):
    big = pl.BlockSpec((1, LQ, 8, 512), lambda c: (c, 0, 0, 0))
    kqs = pl.BlockSpec((1, LQ, SD, 128), lambda c: (c, 0, 0, 0))
    return pl.pallas_call(
        _scan_body,
        grid=(2,),
        in_specs=[big, big, big, kqs, kqs,
                  pl.BlockSpec((1, SD, 512), lambda c: (c, 0, 0)),
                  pl.BlockSpec((16, 512), lambda c: (0, 0))],
        out_specs=[big, pl.BlockSpec((1, SD, 512), lambda c: (c, 0, 0))],
        out_shape=[jax.ShapeDtypeStruct((2, LQ, 8, 512), f32),
                   jax.ShapeDtypeStruct((2, SD, 512), f32)],
        compiler_params=_cp("parallel"),
    )(a8, v8, b8, k8, q8, s0, e16)


def _ctx_body(c_ref, g_ref, h_ref, nw_ref, wo_ref, out_ref):
    ctx = c_ref[...].reshape(TL, SDIM)
    g = g_ref[0]
    x = _rms(ctx, nw_ref[...])
    y = x * (g * jax.nn.sigmoid(g))
    out_ref[0] = h_ref[0] + _mmT(y, wo_ref[...])


def _ctx_call(ctx8, gate, h3d, nw, Wout):
    return pl.pallas_call(
        _ctx_body,
        grid=(B, L // TL),
        in_specs=[pl.BlockSpec((1, TL // 8, 8, SDIM),
                               lambda b, t: (b // 2, t, 0, b % 2)),
                  pl.BlockSpec((1, TL, SDIM), lambda b, t: (b, t, 0)),
                  pl.BlockSpec((1, TL, HID), lambda b, t: (b, t, 0)),
                  pl.BlockSpec((1, SDIM), lambda b, t: (0, 0)),
                  pl.BlockSpec((HID, SDIM), lambda b, t: (0, 0))],
        out_specs=pl.BlockSpec((1, TL, HID), lambda b, t: (b, t, 0)),
        out_shape=jax.ShapeDtypeStruct((B, L, HID), f32),
        compiler_params=_cp("parallel", "parallel"),
    )(ctx8, gate, h3d, nw, Wout)


def _rothalf(x):
    parts = []
    for s0 in range(0, HID, HD):
        parts.append(-x[:, s0 + 32:s0 + 64])
        parts.append(x[:, s0:s0 + 32])
    return jnp.concatenate(parts, axis=1)


def _qkv_body(h_ref, w_ref, ln_ref, cos_ref, sin_ref, q_ref, k_ref, v_ref):
    x = _rms(h_ref[...], ln_ref[...])
    qkv = _mmT(x, w_ref[...])
    q = qkv[:, :HID]; k = qkv[:, HID:2 * HID]; v = qkv[:, 2 * HID:]
    cos = cos_ref[...]; sin = sin_ref[...]
    q_ref[...] = q * cos + _rothalf(q) * sin
    k_ref[...] = k * cos + _rothalf(k) * sin
    v_ref[...] = v


def _qkv_call(h2d, Wqkv, ln, cosT, sinT):
    o = jax.ShapeDtypeStruct((B * L, HID), f32)
    ospec = pl.BlockSpec((TL, HID), lambda g: (g, 0))
    return pl.pallas_call(
        _qkv_body,
        grid=(B * L // TL,),
        in_specs=[pl.BlockSpec((TL, HID), lambda g: (g, 0)),
                  pl.BlockSpec((3 * HID, HID), lambda g: (0, 0)),
                  pl.BlockSpec((1, HID), lambda g: (0, 0)),
                  pl.BlockSpec((TL, HID), lambda g: (g % 2, 0)),
                  pl.BlockSpec((TL, HID), lambda g: (g % 2, 0))],
        out_specs=[ospec] * 3,
        out_shape=[o] * 3,
        compiler_params=_cp("parallel"),
    )(h2d, Wqkv, ln, cosT, sinT)


def _attn_body(q_ref, k_ref, v_ref, m_ref, o_ref):
    outs = []
    for hh in range(2):
        sl = slice(hh * HD, (hh + 1) * HD)
        q = q_ref[0, :, sl] * (1.0 / np.float32(np.sqrt(HD)))
        k = k_ref[0, :, sl]
        v = v_ref[0, :, sl]
        s = _mmT(q, k) + m_ref[...]
        m = jnp.max(s, axis=-1, keepdims=True)
        p = jnp.exp(s - m)
        o = jax.lax.dot_general(p, v, (((1,), (0,)), ((), ())),
                                preferred_element_type=f32)
        outs.append(o / jnp.sum(p, axis=-1, keepdims=True))
    o_ref[0] = jnp.concatenate(outs, axis=1)


def _attn_call(q3d, k3d, v3d, maskb):
    spec = pl.BlockSpec((1, L, 2 * HD), lambda p: (p // 4, 0, p % 4))
    return pl.pallas_call(
        _attn_body,
        grid=(B * NH // 2,),
        in_specs=[spec, spec, spec,
                  pl.BlockSpec((L, L), lambda p: (0, 0))],
        out_specs=spec,
        out_shape=jax.ShapeDtypeStruct((B, L, HID), f32),
        compiler_params=_cp("parallel"),
    )(q3d, k3d, v3d, maskb)


def _ffn_body(h_ref, a_ref, wo_ref, ln_ref, wgu_ref, wd_ref, o_ref):
    h2 = h_ref[...] + _mmT(a_ref[...], wo_ref[...])
    x = _rms(h2, ln_ref[...])
    gu = _mmT(x, wgu_ref[...])
    gg = gu[:, :FFN]; uu = gu[:, FFN:]
    o_ref[...] = h2 + _mmT(gg * jax.nn.sigmoid(gg) * uu, wd_ref[...])


def _ffn_call(h2d, a2d, wo_l, ln, Wgu, wd_l):
    return pl.pallas_call(
        _ffn_body,
        grid=(B * L // TL,),
        in_specs=[pl.BlockSpec((TL, HID), lambda g: (g, 0)),
                  pl.BlockSpec((TL, HID), lambda g: (g, 0)),
                  pl.BlockSpec((HID, HID), lambda g: (0, 0)),
                  pl.BlockSpec((1, HID), lambda g: (0, 0)),
                  pl.BlockSpec((2 * FFN, HID), lambda g: (0, 0)),
                  pl.BlockSpec((HID, FFN), lambda g: (0, 0))],
        out_specs=pl.BlockSpec((TL, HID), lambda g: (g, 0)),
        out_shape=jax.ShapeDtypeStruct((B * L, HID), f32),
        compiler_params=_cp("parallel"),
    )(h2d, a2d, wo_l, ln, Wgu, wd_l)


def _final_body(h_ref, w_ref, emb_ref, o_ref):
    x = _rms(h_ref[...], w_ref[...])
    o_ref[...] = _mmT(x, emb_ref[...])


def _final_call(h2d, fnw, embed_pad):
    return pl.pallas_call(
        _final_body,
        grid=(B * L // TL,),
        in_specs=[pl.BlockSpec((TL, HID), lambda g: (g, 0)),
                  pl.BlockSpec((1, HID), lambda g: (0, 0)),
                  pl.BlockSpec((VOCP, HID), lambda g: (0, 0))],
        out_specs=pl.BlockSpec((TL, VOCP), lambda g: (g, 0)),
        out_shape=jax.ShapeDtypeStruct((B * L, VOCP), f32),
        compiler_params=_cp("parallel"),
    )(h2d, fnw, embed_pad)


def _to8(x):
    y = x.reshape(2, 2, LQ, 8, SDIM)
    return y.transpose(0, 2, 3, 1, 4).reshape(2, LQ, 8, 512)


def _toT8(x):
    y = x.reshape(2, 2, LQ, 8, SH, SD)
    return y.transpose(0, 2, 5, 3, 1, 4).reshape(2, LQ, SD, 128)


def _pack_s(state):
    return state.reshape(2, 2, SH, SD, SD).transpose(0, 4, 1, 2, 3) \
                .reshape(2, SD, 512)


def _unpack_s(sout):
    return sout.reshape(2, SD, 2, SH, SD).transpose(0, 2, 3, 4, 1) \
               .reshape(B, SH, SD, SD)


def kernel(input_ids, state, embed,
           sr_Wk, sr_Wv, sr_Wq, sr_Wbw, sr_Wbb, sr_Waw, sr_Wab, sr_Wout,
           sr_nw, sr_gw,
           sw_Wk, sw_Wv, sw_Wq, sw_Wbw, sw_Wbb, sw_Waw, sw_Wab, sw_Wout,
           sw_nw, sw_gw,
           ln1_w, wq, wk, wv, wo, ln2_w, w_gate, w_up, w_down, final_norm_w):
    cosT = jnp.asarray(_COS_NP)
    sinT = jnp.asarray(_SIN_NP)
    maskb = jnp.asarray(_MASKB_NP)
    mones = jnp.asarray(_MONES_NP)
    e16 = jnp.asarray(_E16_NP)

    ids_f = input_ids.astype(jnp.int32).reshape(B, 1, L)
    embed_pad = jnp.zeros((VOCP, HID), f32).at[:VOC].set(embed)
    h = _embed_call(ids_f, embed_pad).reshape(B * L, HID)

    def delta(h2d, Wk, Wv, Wq, Wbw, Wbb, Waw, Wab, Wout, nw, gw, S0):
        Wcat = jnp.concatenate([Wk, Wv, Wq, Wbw, Waw, gw], axis=0)
        kh, vv, qq, be, al, gg = _proj_call(
            h2d.reshape(B, L, HID), Wcat,
            Wbb.reshape(1, SDIM), Wab.reshape(1, SDIM), mones)
        ctx8, sout = _scan_call(_to8(al), _to8(vv), _to8(be),
                                _toT8(kh), _toT8(qq), _pack_s(S0), e16)
        hn = _ctx_call(ctx8, gg, h2d.reshape(B, L, HID),
                       nw.reshape(1, SDIM), Wout)
        return hn.reshape(B * L, HID), _unpack_s(sout)

    h, S = delta(h, sr_Wk, sr_Wv, sr_Wq, sr_Wbw, sr_Wbb, sr_Waw, sr_Wab,
                 sr_Wout, sr_nw, sr_gw, state)

    for i in range(NL):
        Wqkv = jnp.concatenate([wq[i], wk[i], wv[i]], axis=0)
        q, k, v = _qkv_call(h, Wqkv, ln1_w[i].reshape(1, HID), cosT, sinT)
        a = _attn_call(q.reshape(B, L, HID), k.reshape(B, L, HID),
                       v.reshape(B, L, HID), maskb)
        h = _ffn_call(h, a.reshape(B * L, HID), wo[i],
                      ln2_w[i].reshape(1, HID),
                      jnp.concatenate([w_gate[i], w_up[i]], axis=0),
                      w_down[i])

    h, S = delta(h, sw_Wk, sw_Wv, sw_Wq, sw_Wbw, sw_Wbb, sw_Waw, sw_Wab,
                 sw_Wout, sw_nw, sw_gw, S)

    logits = _final_call(h, final_norm_w.reshape(1, HID), embed_pad)
    return logits[:, :VOC].reshape(B, L, VOC), S
